# baseline (speedup 1.0000x reference)
import functools

import jax
import jax.numpy as jnp
from jax import lax
from jax.experimental import pallas as pl
from jax.experimental.pallas import tpu as pltpu
from jax.experimental.pallas import tpu_sc as plsc

E = 8
K = 2
D = 1024
H = 2048
N = 2048

T = 256
NT = 24
PAD = NT * T
NW = 32
RPW = PAD // NW
RCH = 96
CH = 512


def _router_kernel(x_ref, wr_ref, br_ref,
                   d0_ref, d1_ref, v1_ref, v2_ref, te_ref, ntu_ref,
                   loss_ref):
    x = x_ref[...]
    logits = jnp.dot(x, wr_ref[...], preferred_element_type=jnp.float32)
    logits = logits + br_ref[...][None, :]
    logits = logits - jnp.max(logits, axis=-1, keepdims=True)
    ex = jnp.exp(logits)
    scores = ex / jnp.sum(ex, axis=-1, keepdims=True)

    col = lax.broadcasted_iota(jnp.int32, scores.shape, 1)
    v1 = jnp.max(scores, axis=-1, keepdims=True)
    i1 = jnp.min(jnp.where(scores == v1, col, E), axis=-1, keepdims=True)
    m1 = col == i1
    rest = jnp.where(m1, -jnp.inf, scores)
    v2 = jnp.max(rest, axis=-1, keepdims=True)
    i2 = jnp.min(jnp.where(rest == v2, col, E), axis=-1, keepdims=True)
    m2 = col == i2
    v1_ref[...] = v1
    v2_ref[...] = v2

    M1 = m1.astype(jnp.float32)
    M2 = m2.astype(jnp.float32)

    imp = jnp.sum(M1 * v1 + M2 * v2, axis=0)
    imean = jnp.mean(imp)
    ivar = jnp.sum((imp - imean) ** 2) / (E - 1)
    loss_ref[...] = jnp.reshape(ivar / (imean * imean + 1e-9), (1, 1))

    counts = jnp.sum(M1, axis=0, keepdims=True) + jnp.sum(
        M2, axis=0, keepdims=True)
    ci = counts.astype(jnp.int32)
    pc = ((ci + (T - 1)) >> 8) << 8
    pcf = pc.astype(jnp.float32)
    r8 = lax.broadcasted_iota(jnp.int32, (E, E), 0)
    c8 = lax.broadcasted_iota(jnp.int32, (E, E), 1)
    lt8 = (r8 < c8).astype(jnp.float32)
    starts = jnp.dot(pcf, lt8,
                     preferred_element_type=jnp.float32)
    total = jnp.sum(pc)
    ntu_ref[...] = jnp.reshape(total >> 8, (1, 1))

    ts = (T * lax.broadcasted_iota(jnp.int32, (NT, E), 0)).astype(jnp.float32)
    sb = jnp.broadcast_to(starts, (NT, E))
    pb = jnp.broadcast_to(pcf, (NT, E))
    ind = jnp.logical_and(ts >= sb, ts < sb + pb)
    eidx = lax.broadcasted_iota(jnp.int32, (NT, E), 1)
    te = jnp.sum(jnp.where(ind, eidx + 1, 0), axis=1, keepdims=True) - 1
    te_ref[...] = jnp.where(te < 0, E - 1, te)

    rr = lax.broadcasted_iota(jnp.int32, (CH, CH), 0)
    cc = lax.broadcasted_iota(jnp.int32, (CH, CH), 1)
    ltc = (cc < rr).astype(jnp.float32)
    carry = jnp.zeros((1, E), jnp.float32)
    for c in range(2 * N // CH):
        if c < N // CH:
            Fc = M1[c * CH:(c + 1) * CH]
        else:
            Fc = M2[(c - N // CH) * CH:(c - N // CH + 1) * CH]
        Rc = jnp.dot(ltc, Fc, preferred_element_type=jnp.float32) + carry
        dest = jnp.sum((starts + Rc) * Fc, axis=1, keepdims=True)
        dest = dest.astype(jnp.int32)
        if c < N // CH:
            d0_ref[pl.ds(c * CH, CH), :] = dest
        else:
            d1_ref[pl.ds((c - N // CH) * CH, CH), :] = dest
        carry = carry + jnp.sum(Fc, axis=0, keepdims=True)


def _sc_dispatch_kernel(dest_hbm, x_hbm, xs_hbm, dest_v, rt_v, rows_v, sem):
    wid = lax.axis_index("s") * 2 + lax.axis_index("c")
    lo = wid * RPW

    for i in range(RPW // 16):
        rt_v[pl.ds(i * 16, 16)] = jnp.zeros((16,), jnp.int32)

    pltpu.sync_copy(dest_hbm, dest_v)

    def body(i, _):
        d = dest_v[pl.ds(i * 16, 16)]
        f = lax.iota(jnp.int32, 16) + i * 16
        tok = f & (N - 1)
        m = jnp.logical_and(d >= lo, d < lo + RPW)
        plsc.store_scatter(rt_v, [d - lo], tok, mask=m)
        return _

    lax.fori_loop(0, (K * N) // 16, body, None)

    for c in range(RPW // RCH):
        idx = rt_v.at[pl.ds(c * RCH, RCH)]
        pltpu.async_copy(x_hbm.at[idx], rows_v, sem).wait()
        pltpu.sync_copy(rows_v, xs_hbm.at[pl.ds(lo + c * RCH, RCH)])


def _ffn_kernel(te_ref, ntu_ref, xs_ref, w1_ref, b1_ref, w2_ref, b2_ref,
                d0_ref, d1_ref, v1_ref, v2_ref, out_ref):
    t = pl.program_id(0)

    @pl.when(t == 0)
    def _init():
        out_ref[...] = jnp.zeros_like(out_ref)

    @pl.when(t < ntu_ref[0])
    def _compute():
        xs = xs_ref[...].astype(jnp.bfloat16)
        h = jnp.dot(xs, w1_ref[0].astype(jnp.bfloat16),
                    preferred_element_type=jnp.float32)
        h = jnp.maximum(h + b1_ref[0], 0.0).astype(jnp.bfloat16)
        o = jnp.dot(h, w2_ref[0].astype(jnp.bfloat16),
                    preferred_element_type=jnp.float32)
        o = (o + b2_ref[0]).astype(jnp.bfloat16)

        riota = lax.broadcasted_iota(jnp.int32, (N, T), 1) + t * T
        pc = jnp.where(d0_ref[...] == riota, v1_ref[...], 0.0) + \
            jnp.where(d1_ref[...] == riota, v2_ref[...], 0.0)
        out_ref[...] += jnp.dot(pc.astype(jnp.bfloat16), o,
                                preferred_element_type=jnp.float32)


@functools.cache
def _sc_dispatch():
    return pl.kernel(
        _sc_dispatch_kernel,
        mesh=plsc.VectorSubcoreMesh(core_axis_name="c", subcore_axis_name="s"),
        out_type=jax.ShapeDtypeStruct((PAD, D), jnp.float32),
        scratch_types=[
            pltpu.VMEM((K * N,), jnp.int32),
            pltpu.VMEM((RPW,), jnp.int32),
            pltpu.VMEM((RCH, D), jnp.float32),
            pltpu.SemaphoreType.DMA,
        ],
        compiler_params=pltpu.CompilerParams(needs_layout_passes=False),
    )


@jax.jit
def kernel(x, Wr, br, W1, b1, W2, b2):
    d0, d1, v1, v2, te, ntu, loss = pl.pallas_call(
        _router_kernel,
        out_shape=(
            jax.ShapeDtypeStruct((N, 1), jnp.int32),
            jax.ShapeDtypeStruct((N, 1), jnp.int32),
            jax.ShapeDtypeStruct((N, 1), jnp.float32),
            jax.ShapeDtypeStruct((N, 1), jnp.float32),
            jax.ShapeDtypeStruct((NT, 1), jnp.int32),
            jax.ShapeDtypeStruct((1, 1), jnp.int32),
            jax.ShapeDtypeStruct((1, 1), jnp.float32),
        ),
    )(x, Wr, br)

    dest = jnp.concatenate([d0.reshape(N), d1.reshape(N)])
    xs = _sc_dispatch()(dest, x)

    grid_spec = pltpu.PrefetchScalarGridSpec(
        num_scalar_prefetch=2,
        grid=(NT,),
        in_specs=[
            pl.BlockSpec((T, D), lambda t, te, ntu: (t, 0)),
            pl.BlockSpec((1, D, H), lambda t, te, ntu: (te[t], 0, 0)),
            pl.BlockSpec((1, 1, H), lambda t, te, ntu: (te[t], 0, 0)),
            pl.BlockSpec((1, H, D), lambda t, te, ntu: (te[t], 0, 0)),
            pl.BlockSpec((1, 1, D), lambda t, te, ntu: (te[t], 0, 0)),
            pl.BlockSpec((N, 1), lambda t, te, ntu: (0, 0)),
            pl.BlockSpec((N, 1), lambda t, te, ntu: (0, 0)),
            pl.BlockSpec((N, 1), lambda t, te, ntu: (0, 0)),
            pl.BlockSpec((N, 1), lambda t, te, ntu: (0, 0)),
        ],
        out_specs=pl.BlockSpec((N, D), lambda t, te, ntu: (0, 0)),
    )
    out = pl.pallas_call(
        _ffn_kernel,
        grid_spec=grid_spec,
        out_shape=jax.ShapeDtypeStruct((N, D), jnp.float32),
    )(te.reshape(NT), ntu.reshape(1), xs, W1, b1.reshape(E, 1, H), W2,
      b2.reshape(E, 1, D), d0, d1, v1, v2)

    return out, loss[0, 0]


# speedup vs baseline: 1.8068x; 1.8068x over previous
import functools

import jax
import jax.numpy as jnp
from jax import lax
from jax.experimental import pallas as pl
from jax.experimental.pallas import tpu as pltpu
from jax.experimental.pallas import tpu_sc as plsc

E = 8
K = 2
D = 1024
H = 2048
N = 2048

T = 256
NT = 24
PAD = NT * T
NW = 32
RPW = PAD // NW
RCH = 96
CH = 512


def _router_kernel(x_ref, wr_ref, br_ref,
                   d0_ref, d1_ref, v1_ref, v2_ref, te_ref, ntu_ref,
                   loss_ref):
    x = x_ref[...]
    logits = jnp.dot(x, wr_ref[...], preferred_element_type=jnp.float32)
    logits = logits + br_ref[...][None, :]
    logits = logits - jnp.max(logits, axis=-1, keepdims=True)
    ex = jnp.exp(logits)
    scores = ex / jnp.sum(ex, axis=-1, keepdims=True)

    col = lax.broadcasted_iota(jnp.int32, scores.shape, 1)
    v1 = jnp.max(scores, axis=-1, keepdims=True)
    i1 = jnp.min(jnp.where(scores == v1, col, E), axis=-1, keepdims=True)
    m1 = col == i1
    rest = jnp.where(m1, -jnp.inf, scores)
    v2 = jnp.max(rest, axis=-1, keepdims=True)
    i2 = jnp.min(jnp.where(rest == v2, col, E), axis=-1, keepdims=True)
    m2 = col == i2
    v1_ref[...] = v1
    v2_ref[...] = v2

    M1 = m1.astype(jnp.float32)
    M2 = m2.astype(jnp.float32)

    imp = jnp.sum(M1 * v1 + M2 * v2, axis=0)
    imean = jnp.mean(imp)
    ivar = jnp.sum((imp - imean) ** 2) / (E - 1)
    loss_ref[...] = jnp.reshape(ivar / (imean * imean + 1e-9), (1, 1))

    counts = jnp.sum(M1, axis=0, keepdims=True) + jnp.sum(
        M2, axis=0, keepdims=True)
    ci = counts.astype(jnp.int32)
    pc = ((ci + (T - 1)) >> 8) << 8
    pcf = pc.astype(jnp.float32)
    r8 = lax.broadcasted_iota(jnp.int32, (E, E), 0)
    c8 = lax.broadcasted_iota(jnp.int32, (E, E), 1)
    lt8 = (r8 < c8).astype(jnp.float32)
    starts = jnp.dot(pcf, lt8,
                     preferred_element_type=jnp.float32)
    total = jnp.sum(pc)
    ntu_ref[...] = jnp.reshape(total >> 8, (1, 1))

    ts = (T * lax.broadcasted_iota(jnp.int32, (NT, E), 0)).astype(jnp.float32)
    sb = jnp.broadcast_to(starts, (NT, E))
    pb = jnp.broadcast_to(pcf, (NT, E))
    ind = jnp.logical_and(ts >= sb, ts < sb + pb)
    eidx = lax.broadcasted_iota(jnp.int32, (NT, E), 1)
    te = jnp.sum(jnp.where(ind, eidx + 1, 0), axis=1, keepdims=True) - 1
    te_ref[...] = jnp.where(te < 0, E - 1, te)

    rr = lax.broadcasted_iota(jnp.int32, (CH, CH), 0)
    cc = lax.broadcasted_iota(jnp.int32, (CH, CH), 1)
    ltc = (cc < rr).astype(jnp.float32)
    carry = jnp.zeros((1, E), jnp.float32)
    for c in range(2 * N // CH):
        if c < N // CH:
            Fc = M1[c * CH:(c + 1) * CH]
        else:
            Fc = M2[(c - N // CH) * CH:(c - N // CH + 1) * CH]
        Rc = jnp.dot(ltc, Fc, preferred_element_type=jnp.float32) + carry
        dest = jnp.sum((starts + Rc) * Fc, axis=1, keepdims=True)
        dest = dest.astype(jnp.int32)
        if c < N // CH:
            d0_ref[pl.ds(c * CH, CH), :] = dest
        else:
            d1_ref[pl.ds((c - N // CH) * CH, CH), :] = dest
        carry = carry + jnp.sum(Fc, axis=0, keepdims=True)


def _sc_dispatch_kernel(dest_hbm, x_hbm, xs_hbm, dest_v, rt_v, rows_v, sem):
    wid = lax.axis_index("s") * 2 + lax.axis_index("c")
    lo = wid * RPW

    for i in range(RPW // 16):
        rt_v[pl.ds(i * 16, 16)] = jnp.zeros((16,), jnp.int32)

    pltpu.sync_copy(dest_hbm, dest_v)

    def body(i, _):
        d = dest_v[pl.ds(i * 16, 16)]
        f = lax.iota(jnp.int32, 16) + i * 16
        tok = f & (N - 1)
        m = jnp.logical_and(d >= lo, d < lo + RPW)
        plsc.store_scatter(rt_v, [d - lo], tok, mask=m)
        return _

    lax.fori_loop(0, (K * N) // 16, body, None)

    for c in range(RPW // RCH):
        idx = rt_v.at[pl.ds(c * RCH, RCH)]
        pltpu.async_copy(x_hbm.at[idx], rows_v, sem).wait()
        pltpu.sync_copy(rows_v, xs_hbm.at[pl.ds(lo + c * RCH, RCH)])


def _ffn_kernel(te_ref, ntu_ref, x_ref, w1_ref, b1_ref, w2_ref, b2_ref,
                d0_ref, d1_ref, v1_ref, v2_ref, out_ref):
    t = pl.program_id(0)

    @pl.when(t == 0)
    def _init():
        out_ref[...] = jnp.zeros_like(out_ref)

    @pl.when(t < ntu_ref[0])
    def _compute():
        riota = lax.broadcasted_iota(jnp.int32, (N, T), 1) + t * T
        eq0 = d0_ref[...] == riota
        eq1 = d1_ref[...] == riota

        pgt = (eq0 | eq1).astype(jnp.bfloat16)
        xs = lax.dot_general(
            pgt, x_ref[...].astype(jnp.bfloat16), (((0,), (0,)), ((), ())),
            preferred_element_type=jnp.float32).astype(jnp.bfloat16)

        h = jnp.dot(xs, w1_ref[0].astype(jnp.bfloat16),
                    preferred_element_type=jnp.float32)
        h = jnp.maximum(h + b1_ref[0], 0.0).astype(jnp.bfloat16)
        o = jnp.dot(h, w2_ref[0].astype(jnp.bfloat16),
                    preferred_element_type=jnp.float32)
        o = (o + b2_ref[0]).astype(jnp.bfloat16)

        pc = jnp.where(eq0, v1_ref[...], 0.0) + jnp.where(eq1, v2_ref[...], 0.0)
        out_ref[...] += jnp.dot(pc.astype(jnp.bfloat16), o,
                                preferred_element_type=jnp.float32)


@functools.cache
def _sc_dispatch():
    return pl.kernel(
        _sc_dispatch_kernel,
        mesh=plsc.VectorSubcoreMesh(core_axis_name="c", subcore_axis_name="s"),
        out_type=jax.ShapeDtypeStruct((PAD, D), jnp.float32),
        scratch_types=[
            pltpu.VMEM((K * N,), jnp.int32),
            pltpu.VMEM((RPW,), jnp.int32),
            pltpu.VMEM((RCH, D), jnp.float32),
            pltpu.SemaphoreType.DMA,
        ],
        compiler_params=pltpu.CompilerParams(needs_layout_passes=False),
    )


@jax.jit
def kernel(x, Wr, br, W1, b1, W2, b2):
    d0, d1, v1, v2, te, ntu, loss = pl.pallas_call(
        _router_kernel,
        out_shape=(
            jax.ShapeDtypeStruct((N, 1), jnp.int32),
            jax.ShapeDtypeStruct((N, 1), jnp.int32),
            jax.ShapeDtypeStruct((N, 1), jnp.float32),
            jax.ShapeDtypeStruct((N, 1), jnp.float32),
            jax.ShapeDtypeStruct((NT, 1), jnp.int32),
            jax.ShapeDtypeStruct((1, 1), jnp.int32),
            jax.ShapeDtypeStruct((1, 1), jnp.float32),
        ),
    )(x, Wr, br)

    grid_spec = pltpu.PrefetchScalarGridSpec(
        num_scalar_prefetch=2,
        grid=(NT,),
        in_specs=[
            pl.BlockSpec((N, D), lambda t, te, ntu: (0, 0)),
            pl.BlockSpec((1, D, H), lambda t, te, ntu: (te[t], 0, 0)),
            pl.BlockSpec((1, 1, H), lambda t, te, ntu: (te[t], 0, 0)),
            pl.BlockSpec((1, H, D), lambda t, te, ntu: (te[t], 0, 0)),
            pl.BlockSpec((1, 1, D), lambda t, te, ntu: (te[t], 0, 0)),
            pl.BlockSpec((N, 1), lambda t, te, ntu: (0, 0)),
            pl.BlockSpec((N, 1), lambda t, te, ntu: (0, 0)),
            pl.BlockSpec((N, 1), lambda t, te, ntu: (0, 0)),
            pl.BlockSpec((N, 1), lambda t, te, ntu: (0, 0)),
        ],
        out_specs=pl.BlockSpec((N, D), lambda t, te, ntu: (0, 0)),
    )
    out = pl.pallas_call(
        _ffn_kernel,
        grid_spec=grid_spec,
        out_shape=jax.ShapeDtypeStruct((N, D), jnp.float32),
        compiler_params=pltpu.CompilerParams(
            fuse_transposed_lhs_in_matmul=True,
            vmem_limit_bytes=128 * 1024 * 1024),
    )(te.reshape(NT), ntu.reshape(1), x, W1, b1.reshape(E, 1, H), W2,
      b2.reshape(E, 1, D), d0, d1, v1, v2)

    return out, loss[0, 0]
